# Initial kernel scaffold; baseline (speedup 1.0000x reference)
import jax
import jax.numpy as jnp
from jax.experimental import pallas as pl


def kernel(x, edge_index, edge_attr, k, W1, b1, W2, b2, W3, b3):
    raise NotImplementedError("write your pallas kernel here")


# speedup vs baseline: 11.6462x; 11.6462x over previous
import functools

import numpy as np

import jax
import jax.numpy as jnp
from jax import lax
from jax.experimental import pallas as pl
from jax.experimental.pallas import tpu as pltpu
from jax.experimental.pallas import tpu_sc as plsc

N = 100000
NPAD = 100096
E = 3200000
EROWS = 25000
NW = 32
RS = NPAD // 16
F2R = 782
RB = RS // 128
REM = RS - RB * 128

_mesh = plsc.VectorSubcoreMesh(
    core_axis_name="c", subcore_axis_name="s", num_cores=2, num_subcores=16)

_i32 = jnp.int32


def _iota16():
    return lax.iota(_i32, 16)


def _worker_bounds(wid):
    lo = (EROWS * wid) // NW
    hi = (EROWS * (wid + 1)) // NW
    return lo, hi


def _fill_consec(idxb, base, nvec):
    for j in range(nvec):
        idxb[pl.ds(16 * j, 16)] = _iota16() + base + _i32(16 * j)


def _make_deg():
    @functools.partial(
        pl.kernel,
        out_type=jax.ShapeDtypeStruct((2 * NPAD,), jnp.float32),
        mesh=_mesh,
        scratch_types=[
            pltpu.VMEM_SHARED((NPAD,), jnp.float32),
            pltpu.VMEM((128,), _i32),
            pltpu.VMEM((128,), jnp.float32),
            pltpu.VMEM((128,), jnp.float32),
            pltpu.VMEM((RS,), jnp.float32),
        ],
    )
    def deg(col1, ea1, zeros_hbm, outp, acc, colb, eab, vals, stage):
        cid = lax.axis_index("c")
        sid = lax.axis_index("s")
        wid = cid * 16 + sid
        r0 = sid * RS
        pltpu.sync_copy(zeros_hbm.at[pl.ds(r0, RS)], stage)
        pltpu.sync_copy(stage, acc.at[pl.ds(r0, RS)])
        plsc.subcore_barrier()
        lo, hi = _worker_bounds(wid)

        def body(j, carry):
            pltpu.sync_copy(col1.at[pl.ds(j * 128, 128)], colb)
            pltpu.sync_copy(ea1.at[pl.ds(j * 128, 128)], eab)
            for i in range(8):
                vals[pl.ds(16 * i, 16)] = jnp.abs(eab[pl.ds(16 * i, 16)])
            pltpu.sync_copy(vals, acc.at[colb], add=True)
            return carry

        lax.fori_loop(lo, hi, body, 0)
        plsc.subcore_barrier()
        pltpu.sync_copy(acc.at[pl.ds(r0, RS)], stage)
        pltpu.sync_copy(stage, outp.at[pl.ds(cid * NPAD + r0, RS)])

    return deg


def _make_norm():
    @functools.partial(
        pl.kernel,
        out_type=jax.ShapeDtypeStruct((E,), jnp.float32),
        mesh=_mesh,
        scratch_types=[
            pltpu.VMEM_SHARED((NPAD,), jnp.float32),
            pltpu.VMEM((128,), _i32),
            pltpu.VMEM((128,), _i32),
            pltpu.VMEM((128,), jnp.float32),
            pltpu.VMEM((128,), jnp.float32),
            pltpu.VMEM((128,), jnp.float32),
            pltpu.VMEM((128,), jnp.float32),
            pltpu.VMEM((RS,), jnp.float32),
        ],
    )
    def norm(dis_hbm, row1, col1, ea1, nw1, disS, rowb, colb, eab, dr, dc, nwb,
             stage):
        cid = lax.axis_index("c")
        sid = lax.axis_index("s")
        wid = cid * 16 + sid
        r0 = sid * RS
        pltpu.sync_copy(dis_hbm.at[pl.ds(r0, RS)], stage)
        pltpu.sync_copy(stage, disS.at[pl.ds(r0, RS)])
        plsc.subcore_barrier()
        lo, hi = _worker_bounds(wid)

        def body(j, carry):
            js = pl.ds(j * 128, 128)
            pltpu.sync_copy(row1.at[js], rowb)
            pltpu.sync_copy(col1.at[js], colb)
            pltpu.sync_copy(ea1.at[js], eab)
            pltpu.sync_copy(disS.at[rowb], dr)
            pltpu.sync_copy(disS.at[colb], dc)
            for i in range(8):
                sl = pl.ds(16 * i, 16)
                nwb[sl] = dr[sl] * dc[sl] * jnp.abs(eab[sl])
            pltpu.sync_copy(nwb, nw1.at[js])
            return carry

        lax.fori_loop(lo, hi, body, 0)

    return norm


def _make_hop(f, emit_combined):
    outs = [jax.ShapeDtypeStruct((2 * f * NPAD,), jnp.float32)]
    if emit_combined:
        outs.append(jax.ShapeDtypeStruct((f * NPAD,), jnp.float32))

    scratch = (
        [pltpu.VMEM_SHARED((NPAD,), jnp.float32) for _ in range(f)]
        + [pltpu.VMEM_SHARED((NPAD,), jnp.float32) for _ in range(f)]
        + [pltpu.VMEM((128,), jnp.float32) for _ in range(f)]
        + [
            pltpu.VMEM((128,), _i32),
            pltpu.VMEM((REM,), _i32),
            pltpu.VMEM((128,), jnp.float32),
            pltpu.VMEM((REM,), jnp.float32),
            pltpu.VMEM((128,), _i32),
            pltpu.VMEM((128,), _i32),
            pltpu.VMEM((128,), jnp.float32),
            pltpu.VMEM((RS,), jnp.float32),
        ]
    )

    @functools.partial(
        pl.kernel,
        out_type=tuple(outs) if emit_combined else outs[0],
        mesh=_mesh,
        scratch_types=scratch,
    )
    def hop(pA, pB, zeros_hbm, row1, col1, nw1, *refs):
        if emit_combined:
            outp, comb = refs[0], refs[1]
            rest = refs[2:]
        else:
            outp = refs[0]
            comb = None
            rest = refs[1:]
        tab = rest[0:f]
        acc = rest[f:2 * f]
        vals = rest[2 * f:3 * f]
        idxc, idxr, valb, valr, rowb, colb, wb, stage = rest[3 * f:]

        cid = lax.axis_index("c")
        sid = lax.axis_index("s")
        wid = cid * 16 + sid
        r0 = sid * RS
        sl0 = pl.ds(r0, RS)
        pltpu.sync_copy(zeros_hbm.at[sl0], stage)
        for c in range(f):
            pltpu.sync_copy(stage, acc[c].at[sl0])
        for c in range(f):
            pltpu.sync_copy(pA.at[pl.ds(c * NPAD + r0, RS)], stage)
            pltpu.sync_copy(stage, tab[c].at[sl0])

        def comb_body(b, carry):
            base = r0 + b * 128
            _fill_consec(idxc, base, 8)
            for c in range(f):
                pltpu.sync_copy(pB.at[pl.ds(c * NPAD + base, 128)], valb)
                pltpu.sync_copy(valb, tab[c].at[idxc], add=True)
            return carry

        lax.fori_loop(0, RB, comb_body, 0)
        rbase = r0 + RB * 128
        _fill_consec(idxr, rbase, REM // 16)
        for c in range(f):
            pltpu.sync_copy(pB.at[pl.ds(c * NPAD + rbase, REM)], valr)
            pltpu.sync_copy(valr, tab[c].at[idxr], add=True)
        if emit_combined:
            @pl.when(cid == 0)
            def _():
                for c in range(f):
                    pltpu.sync_copy(tab[c].at[sl0], stage)
                    pltpu.sync_copy(stage, comb.at[pl.ds(c * NPAD + r0, RS)])
        plsc.subcore_barrier()
        lo, hi = _worker_bounds(wid)

        def body(j, carry):
            js = pl.ds(j * 128, 128)
            pltpu.sync_copy(row1.at[js], rowb)
            pltpu.sync_copy(col1.at[js], colb)
            pltpu.sync_copy(nw1.at[js], wb)
            for c in range(f):
                pltpu.sync_copy(tab[c].at[rowb], vals[c])
            for c in range(f):
                for i in range(8):
                    sl = pl.ds(16 * i, 16)
                    vals[c][sl] = vals[c][sl] * wb[sl]
            for c in range(f):
                pltpu.sync_copy(vals[c], acc[c].at[colb], add=True)
            return carry

        lax.fori_loop(lo, hi, body, 0)
        plsc.subcore_barrier()
        for c in range(f):
            pltpu.sync_copy(acc[c].at[sl0], stage)
            pltpu.sync_copy(
                stage, outp.at[pl.ds((cid * f + c) * NPAD + r0, RS)])

    return hop


_deg_k = _make_deg()
_norm_k = _make_norm()
_hop1c = _make_hop(1, True)
_hop1 = _make_hop(1, False)
_hop4c = _make_hop(4, True)
_hop4 = _make_hop(4, False)
_hop8c = _make_hop(8, True)
_hop8 = _make_hop(8, False)


def _dis_body(degp_ref, out_ref):
    deg = degp_ref[0:1, :] + degp_ref[1:2, :]
    safe = jnp.where(deg > 0, deg, 1.0)
    out_ref[...] = jnp.where(deg > 0, lax.rsqrt(safe), 0.0)


def _dis_tc(degp):
    return pl.pallas_call(
        _dis_body,
        out_shape=jax.ShapeDtypeStruct((1, NPAD), jnp.float32),
    )(degp)


def _proj1_body(x1, t1, t2, t3a, t3b, w_ref, b_ref, out_ref):
    t3 = t3a[...] + t3b[...]
    xv = x1[...]
    t1v = t1[...]
    t2v = t2[...]
    for j in range(8):
        o = xv * w_ref[0, j] + b_ref[0, j]
        o = o + t1v * w_ref[1, j]
        o = o + t2v * w_ref[2, j]
        o = o + t3 * w_ref[3, j]
        out_ref[pl.ds(j, 1), :] = jnp.maximum(o, 0.0)


def _proj1_tc(x1, t1, t2, t3a, t3b, w48, b18):
    return pl.pallas_call(
        _proj1_body,
        in_specs=[pl.BlockSpec((1, NPAD), lambda: (0, 0))] * 5
        + [pl.BlockSpec(memory_space=pltpu.SMEM)] * 2,
        out_specs=pl.BlockSpec((8, NPAD), lambda: (0, 0)),
        out_shape=jax.ShapeDtypeStruct((8, NPAD), jnp.float32),
    )(x1, t1, t2, t3a, t3b, w48, b18)


_bf16 = jnp.bfloat16


def _proj2_body(x2, u1, u2, u3a, u3b, w_ref, b_ref, o_ref):
    u3 = u3a[...] + u3b[...]

    def bdot(i, h):
        return jnp.dot(w_ref[pl.ds(8 * i, 8), :].astype(_bf16),
                       h.astype(_bf16), preferred_element_type=jnp.float32)

    o = bdot(0, x2[...]) + b_ref[...]
    o = o + bdot(1, u1[...])
    o = o + bdot(2, u2[...])
    o = o + bdot(3, u3)
    o_ref[...] = jnp.maximum(o, 0.0)


def _proj2_tc(x2, u1, u2, u3a, u3b, wT4, bcol):
    return pl.pallas_call(
        _proj2_body,
        out_shape=jax.ShapeDtypeStruct((8, NPAD), jnp.float32),
    )(x2, u1, u2, u3a, u3b, wT4, bcol)


def _f1_body(x3, c1, c2, c3a, c3b, w3_ref, b3_ref, out_ref):
    c3 = c3a[...] + c3b[...]

    def bdot(i, h):
        return jnp.dot(w3_ref[pl.ds(i, 1), :].astype(_bf16),
                       h.astype(_bf16), preferred_element_type=jnp.float32)

    o = bdot(0, x3[...]) + b3_ref[...]
    o = o + bdot(1, c1[...])
    o = o + bdot(2, c2[...])
    o = o + bdot(3, c3)
    out_ref[...] = jnp.maximum(o, 0.0)


def _f1_tc(x3, c1, c2, c3a, c3b, w3r, b3s):
    return pl.pallas_call(
        _f1_body,
        out_shape=jax.ShapeDtypeStruct((1, NPAD), jnp.float32),
    )(x3, c1, c2, c3a, c3b, w3r, b3s)


_IMIN = np.int32(-(2 ** 31))
_IMAXP = np.int32(0x7FFFFFFF)


def _f2_body(hp_ref, k_ref, hn_ref, soft_ref):
    hp = hp_ref[...]
    ridx = lax.broadcasted_iota(_i32, (F2R, 128), 0)
    cidx = lax.broadcasted_iota(_i32, (F2R, 128), 1)
    node = ridx * 128 + cidx
    valid = node < N
    kk = k_ref[0]

    hsum = jnp.sum(jnp.where(valid, hp, 0.0))
    mean = hsum / N
    dev = jnp.where(valid, hp - mean, 0.0)
    var = jnp.sum(dev * dev) / N
    hn = (hp - mean) / jnp.sqrt(var + 1e-5)

    b = lax.bitcast_convert_type(hn, _i32)
    s = jnp.where(b >= 0, b, b ^ _IMAXP)
    s = jnp.where(valid, s, _IMIN)

    cnt_nn = jnp.sum(jnp.where(s >= 0, 1, 0).astype(_i32))
    pos_branch = cnt_nn >= kk
    mp = jnp.where(s >= 0, s, _i32(-1))
    mn = jnp.where(valid, jnp.where(s < 0, s ^ _IMIN, _IMAXP), _i32(-1))
    m = jnp.where(pos_branch, mp, mn)

    def bit_body(i, v):
        cand = v | lax.shift_left(_i32(1), _i32(30) - i)
        cnt = jnp.sum(jnp.where(m >= cand, 1, 0).astype(_i32))
        return jnp.where(cnt >= kk, cand, v)

    v = lax.fori_loop(0, 31, bit_body, _i32(0))
    u = jnp.where(pos_branch, v, v ^ _IMIN)

    gt = s > u
    eq = jnp.logical_and(s == u, valid)
    cnt_gt = jnp.sum(gt.astype(_i32))
    need = (kk - cnt_gt).astype(jnp.float32)

    eqf = eq.astype(jnp.float32)
    colc = lax.broadcasted_iota(_i32, (128, 128), 0)
    colr = lax.broadcasted_iota(_i32, (128, 128), 1)
    upper = (colc < colr).astype(jnp.float32)
    rowp = jnp.dot(eqf, upper, preferred_element_type=jnp.float32)
    rsum = jnp.sum(eqf, axis=1, keepdims=True)
    li = lax.broadcasted_iota(_i32, (F2R, F2R), 0)
    lj = lax.broadcasted_iota(_i32, (F2R, F2R), 1)
    ltri = (lj < li).astype(jnp.float32)
    carry = jnp.dot(ltri, rsum, preferred_element_type=jnp.float32)
    rank = carry + rowp
    member = jnp.logical_or(gt, jnp.logical_and(eq, rank < need))

    tb = jnp.where(u >= 0, u, u ^ _IMAXP)
    tf = lax.bitcast_convert_type(tb, jnp.float32)
    ka = jnp.abs(tf)
    z = hn - ka + jnp.where(member, 0.1, -0.1)
    hn_ref[...] = hn
    soft_ref[...] = jax.nn.sigmoid(z)


def _f2_tc(hp2d, kvec):
    return pl.pallas_call(
        _f2_body,
        in_specs=[pl.BlockSpec((F2R, 128), lambda: (0, 0)),
                  pl.BlockSpec(memory_space=pltpu.SMEM)],
        out_specs=(pl.BlockSpec((F2R, 128), lambda: (0, 0)),
                   pl.BlockSpec((F2R, 128), lambda: (0, 0))),
        out_shape=(jax.ShapeDtypeStruct((F2R, 128), jnp.float32),
                   jax.ShapeDtypeStruct((F2R, 128), jnp.float32)),
    )(hp2d, kvec)


def kernel(x, edge_index, edge_attr, k, W1, b1, W2, b2, W3, b3):
    f32 = jnp.float32
    row1 = edge_index[0]
    col1 = edge_index[1]
    ea1 = edge_attr
    x1 = jnp.pad(x, (0, NPAD - N)).reshape(1, NPAD)
    zN = jnp.zeros((NPAD,), f32)
    zf = {1: zN, 4: jnp.zeros((4 * NPAD,), f32), 8: jnp.zeros((8 * NPAD,), f32)}

    def hop(fn, f, pA, pB):
        r = fn(pA.reshape(-1), pB.reshape(-1), zN, row1, col1, nw1)
        if isinstance(r, tuple):
            return r[0].reshape(2, f, NPAD), r[1].reshape(f, NPAD)
        return r.reshape(2, f, NPAD)

    degp = _deg_k(col1, ea1, zN)
    dis = _dis_tc(degp.reshape(2, NPAD)).reshape(NPAD)
    nw1 = _norm_k(dis, row1, col1, ea1)

    t1p = hop(_hop1, 1, x1, zf[1])
    t2p, t1 = hop(_hop1c, 1, t1p[0], t1p[1])
    t3p, t2 = hop(_hop1c, 1, t2p[0], t2p[1])
    out1 = _proj1_tc(x1, t1, t2, t3p[0], t3p[1], W1.reshape(4, 8),
                     b1.reshape(1, 8))

    u1p = hop(_hop8, 8, out1, zf[8])
    u2p, u1 = hop(_hop8c, 8, u1p[0], u1p[1])
    u3p, u2 = hop(_hop8c, 8, u2p[0], u2p[1])
    W2T4 = jnp.concatenate([W2[i].T for i in range(4)], axis=0)
    out2 = _proj2_tc(out1, u1, u2, u3p[0], u3p[1], W2T4, b2.reshape(8, 1))

    c1p = hop(_hop8, 8, out2, zf[8])
    c2p, c1 = hop(_hop8c, 8, c1p[0], c1p[1])
    c3p, c2 = hop(_hop8c, 8, c2p[0], c2p[1])

    hp = _f1_tc(out2, c1, c2, c3p[0], c3p[1], W3.reshape(4, 8),
                b3.reshape(1, 1))
    kvec = jnp.asarray(k, _i32).reshape(1)
    hn2d, soft2d = _f2_tc(hp.reshape(F2R, 128), kvec)
    hn = hn2d.reshape(NPAD)[:N]
    soft = soft2d.reshape(NPAD)[:N]
    return jnp.column_stack((hn, soft))
